# baseline (speedup 1.0000x reference)
import functools

import jax
import jax.numpy as jnp
from jax import lax
from jax.experimental import pallas as pl
from jax.experimental.pallas import tpu as pltpu
from jax.experimental.pallas import tpu_sc as plsc

NUM_ENTITIES = 100000
NUM_RELATIONS = 1000
DIM = 200
BATCH = 16384

NC, NS, L = 2, 16, 16
NW = NC * NS
B_PER_W = BATCH // NW
CHUNK = 32
NCHUNK = B_PER_W // CHUNK
ACC_STRIDE = L + 1
SEG = 184

_GROUPS = [
    (0, 0, 1, 72, False), (0, 16, 1, 88, False), (0, 32, 1, 104, False),
    (0, 40, 1, 112, True),
    (0, 56, 2, 0, False), (0, 72, 2, 16, False), (0, 88, 2, 32, False),
    (0, 104, 2, 48, False), (0, 112, 2, 56, True),
    (1, 0, 2, 72, False), (1, 16, 2, 88, False), (1, 32, 2, 104, False),
    (1, 40, 2, 112, True),
]


def _transpose_reduce(accbuf, out_v, out_base, lane, count):
    for k in range(0, count, L):
        base_idx = (lane + k) * ACC_STRIDE
        tot = jnp.zeros((L,), jnp.float32)
        for j in range(L):
            tot = tot + plsc.load_gather(accbuf, [base_idx + j])
        out_v[pl.ds(out_base + k, L)] = tot


def _main_kernel(heads_hbm, rels_hbm, tails_hbm, ent_hbm, rel_hbm,
                 out_hbm, idx_s, bufs_s, accbuf, out_v, sems):
    wid = lax.axis_index("s") * NC + lax.axis_index("c")
    base = wid * B_PER_W
    lane = lax.iota(jnp.int32, L)
    half_mask = lane >= L // 2
    srcs = (heads_hbm, tails_hbm, rels_hbm)

    tables = (ent_hbm, ent_hbm, rel_hbm)

    def copies(slot):
        idxs, bufs = idx_s[slot], bufs_s[slot]
        return [
            pltpu.make_async_copy(
                tables[t].at[idxs[t], pl.ds(128 * k, 128)],
                bufs[3 * t + k], sems[slot])
            for t in range(3) for k in range(3)
        ]

    def fire(g, slot):
        cbase = base + g * CHUNK
        for t in range(3):
            pltpu.sync_copy(srcs[t].at[pl.ds(cbase, CHUNK)], idx_s[slot][t])
        for c in copies(slot):
            c.start()

    def drain(slot):
        for c in copies(slot):
            c.wait()

    def score(g, slot):
        bufs = bufs_s[slot]

        def body(i, carry2):
            acc = jnp.zeros((L,), jnp.float32)
            for re_p, re_o, im_p, im_o, masked in _GROUPS:
                h_re = bufs[re_p][i, pl.ds(re_o, L)]
                h_im = bufs[im_p][i, pl.ds(im_o, L)]
                t_re = bufs[3 + re_p][i, pl.ds(re_o, L)]
                t_im = bufs[3 + im_p][i, pl.ds(im_o, L)]
                r_re = bufs[6 + re_p][i, pl.ds(re_o, L)]
                r_im = bufs[6 + im_p][i, pl.ds(im_o, L)]
                p = h_re * t_re + h_im * t_im
                q = h_re * t_im - h_im * t_re
                term = r_re * p + r_im * q
                if masked:
                    term = jnp.where(half_mask, term, 0.0)
                acc = acc + term
            accbuf[pl.ds(i * ACC_STRIDE, L)] = acc
            return carry2

        lax.fori_loop(0, CHUNK, body, 0)
        _transpose_reduce(accbuf, out_v, g * CHUNK, lane, CHUNK)

    fire(0, 0)

    def pair_body(g2, carry):
        g = 2 * g2
        drain(0)
        fire(g + 1, 1)
        score(g, 0)
        drain(1)

        @pl.when(g + 2 < NCHUNK)
        def _():
            fire(g + 2, 0)

        score(g + 1, 1)
        return carry

    lax.fori_loop(0, NCHUNK // 2, pair_body, 0)
    pltpu.sync_copy(out_v, out_hbm.at[pl.ds(base, B_PER_W)])


def _seg_kernel(heads_hbm, rels_hbm, tails_hbm, ent_seg_hbm, rel_seg_hbm,
                out_hbm, idx_h, idx_t, idx_r, rows_h, rows_t, rows_r,
                accbuf, out_v, sem):
    wid = lax.axis_index("s") * NC + lax.axis_index("c")
    base = wid * B_PER_W
    lane = lax.iota(jnp.int32, L)

    pltpu.sync_copy(heads_hbm.at[pl.ds(base, B_PER_W)], idx_h)
    pltpu.sync_copy(tails_hbm.at[pl.ds(base, B_PER_W)], idx_t)
    pltpu.sync_copy(rels_hbm.at[pl.ds(base, B_PER_W)], idx_r)
    c1 = pltpu.async_copy(ent_seg_hbm.at[idx_h], rows_h, sem)
    c2 = pltpu.async_copy(ent_seg_hbm.at[idx_t], rows_t, sem)
    c3 = pltpu.async_copy(rel_seg_hbm.at[idx_r], rows_r, sem)
    c1.wait()
    c2.wait()
    c3.wait()

    def body(i, carry):
        h_re = rows_h[i, pl.ds(0, L)]
        h_im = rows_h[i, pl.ds(L, L)]
        t_re = rows_t[i, pl.ds(0, L)]
        t_im = rows_t[i, pl.ds(L, L)]
        r_re = rows_r[i, pl.ds(0, L)]
        r_im = rows_r[i, pl.ds(L, L)]
        p = h_re * t_re + h_im * t_im
        q = h_re * t_im - h_im * t_re
        accbuf[pl.ds(i * ACC_STRIDE, L)] = r_re * p + r_im * q
        return carry

    lax.fori_loop(0, B_PER_W, body, 0)
    _transpose_reduce(accbuf, out_v, 0, lane, B_PER_W)
    pltpu.sync_copy(out_v, out_hbm.at[pl.ds(base, B_PER_W)])


@jax.jit
def _compl_ex(heads, rels, tails, entity_emb, rel_emb):
    mesh = plsc.VectorSubcoreMesh(
        core_axis_name="c", subcore_axis_name="s", num_cores=NC,
        num_subcores=NS)
    main = functools.partial(
        pl.kernel,
        out_type=jax.ShapeDtypeStruct((BATCH,), jnp.float32),
        mesh=mesh,
        compiler_params=pltpu.CompilerParams(needs_layout_passes=False),
        scratch_types=[
            [[pltpu.VMEM((CHUNK,), jnp.int32) for _ in range(3)]
             for _ in range(2)],
            [[pltpu.VMEM((CHUNK, 128), jnp.float32) for _ in range(9)]
             for _ in range(2)],
            pltpu.VMEM((CHUNK * ACC_STRIDE,), jnp.float32),
            pltpu.VMEM((B_PER_W,), jnp.float32),
            [pltpu.SemaphoreType.DMA for _ in range(2)],
        ],
    )(_main_kernel)
    seg = functools.partial(
        pl.kernel,
        out_type=jax.ShapeDtypeStruct((BATCH,), jnp.float32),
        mesh=mesh,
        compiler_params=pltpu.CompilerParams(
            needs_layout_passes=False, use_tc_tiling_on_sc=False),
        scratch_types=[
            pltpu.VMEM((B_PER_W,), jnp.int32),
            pltpu.VMEM((B_PER_W,), jnp.int32),
            pltpu.VMEM((B_PER_W,), jnp.int32),
            pltpu.VMEM((B_PER_W, 2 * L), jnp.float32),
            pltpu.VMEM((B_PER_W, 2 * L), jnp.float32),
            pltpu.VMEM((B_PER_W, 2 * L), jnp.float32),
            pltpu.VMEM((B_PER_W * ACC_STRIDE,), jnp.float32),
            pltpu.VMEM((B_PER_W,), jnp.float32),
            pltpu.SemaphoreType.DMA,
        ],
    )(_seg_kernel)
    ent_seg = jnp.concatenate(
        [entity_emb[:, SEG:DIM], entity_emb[:, DIM + SEG:]], axis=1)
    rel_seg = jnp.concatenate(
        [rel_emb[:, SEG:DIM], rel_emb[:, DIM + SEG:]], axis=1)
    part1 = main(heads, rels, tails, entity_emb, rel_emb)
    part2 = seg(heads, rels, tails, ent_seg, rel_seg)
    return part1 + part2


def kernel(heads, rels, tails, entity_emb, rel_emb):
    return _compl_ex(
        heads.astype(jnp.int32),
        rels.astype(jnp.int32),
        tails.astype(jnp.int32),
        entity_emb.astype(jnp.float32),
        rel_emb.astype(jnp.float32),
    )


# speedup vs baseline: 1.0679x; 1.0679x over previous
import functools

import jax
import jax.numpy as jnp
from jax import lax
from jax.experimental import pallas as pl
from jax.experimental.pallas import tpu as pltpu
from jax.experimental.pallas import tpu_sc as plsc

NUM_ENTITIES = 100000
NUM_RELATIONS = 1000
DIM = 200
BATCH = 16384

NC, NS, L = 2, 16, 16
NW = NC * NS
B_PER_W = BATCH // NW
CHUNK = 32
NCHUNK = B_PER_W // CHUNK
ACC_STRIDE = L + 1
SEG = 184

_GROUPS = [
    (0, 0, 1, 72, False), (0, 16, 1, 88, False), (0, 32, 1, 104, False),
    (0, 40, 1, 112, True),
    (0, 56, 2, 0, False), (0, 72, 2, 16, False), (0, 88, 2, 32, False),
    (0, 104, 2, 48, False), (0, 112, 2, 56, True),
    (1, 0, 2, 72, False), (1, 16, 2, 88, False), (1, 32, 2, 104, False),
    (1, 40, 2, 112, True),
]


def _transpose_reduce(accbuf, out_v, out_base, lane, count):
    for k in range(0, count, L):
        base_idx = (lane + k) * ACC_STRIDE
        tot = jnp.zeros((L,), jnp.float32)
        for j in range(L):
            tot = tot + plsc.load_gather(accbuf, [base_idx + j])
        out_v[pl.ds(out_base + k, L)] = tot


def _main_kernel(heads_hbm, rels_hbm, tails_hbm, ent_hbm, rel_hbm,
                 out_hbm, idx_all, bufs_s, accbuf, out_v, sems):
    wid = lax.axis_index("s") * NC + lax.axis_index("c")
    base = wid * B_PER_W
    lane = lax.iota(jnp.int32, L)
    half_mask = lane >= L // 2
    srcs = (heads_hbm, tails_hbm, rels_hbm)
    tables = (ent_hbm, ent_hbm, rel_hbm)

    for t in range(3):
        pltpu.sync_copy(srcs[t].at[pl.ds(base, B_PER_W)], idx_all[t])

    def copies(g, slot):
        bufs = bufs_s[slot]
        return [
            pltpu.make_async_copy(
                tables[t].at[idx_all[t].at[pl.ds(g * CHUNK, CHUNK)],
                             pl.ds(128 * k, 128)],
                bufs[3 * t + k], sems[slot])
            for t in range(3) for k in range(3)
        ]

    def fire(g, slot):
        for c in copies(g, slot):
            c.start()

    def drain(g, slot):
        for c in copies(g, slot):
            c.wait()

    def score(g, slot):
        bufs = bufs_s[slot]

        def body(i, carry2):
            acc = jnp.zeros((L,), jnp.float32)
            for re_p, re_o, im_p, im_o, masked in _GROUPS:
                h_re = bufs[re_p][i, pl.ds(re_o, L)]
                h_im = bufs[im_p][i, pl.ds(im_o, L)]
                t_re = bufs[3 + re_p][i, pl.ds(re_o, L)]
                t_im = bufs[3 + im_p][i, pl.ds(im_o, L)]
                r_re = bufs[6 + re_p][i, pl.ds(re_o, L)]
                r_im = bufs[6 + im_p][i, pl.ds(im_o, L)]
                p = h_re * t_re + h_im * t_im
                q = h_re * t_im - h_im * t_re
                term = r_re * p + r_im * q
                if masked:
                    term = jnp.where(half_mask, term, 0.0)
                acc = acc + term
            accbuf[pl.ds(i * ACC_STRIDE, L)] = acc
            return carry2

        lax.fori_loop(0, CHUNK, body, 0)
        _transpose_reduce(accbuf, out_v, g * CHUNK, lane, CHUNK)

    fire(0, 0)

    def pair_body(g2, carry):
        g = 2 * g2
        drain(g, 0)
        fire(g + 1, 1)
        score(g, 0)
        drain(g + 1, 1)

        @pl.when(g + 2 < NCHUNK)
        def _():
            fire(g + 2, 0)

        score(g + 1, 1)
        return carry

    lax.fori_loop(0, NCHUNK // 2, pair_body, 0)
    pltpu.sync_copy(out_v, out_hbm.at[pl.ds(base, B_PER_W)])


def _seg_kernel(heads_hbm, rels_hbm, tails_hbm, ent_seg_hbm, rel_seg_hbm,
                out_hbm, idx_h, idx_t, idx_r, rows_h, rows_t, rows_r,
                accbuf, out_v, sem):
    wid = lax.axis_index("s") * NC + lax.axis_index("c")
    base = wid * B_PER_W
    lane = lax.iota(jnp.int32, L)

    pltpu.sync_copy(heads_hbm.at[pl.ds(base, B_PER_W)], idx_h)
    pltpu.sync_copy(tails_hbm.at[pl.ds(base, B_PER_W)], idx_t)
    pltpu.sync_copy(rels_hbm.at[pl.ds(base, B_PER_W)], idx_r)
    c1 = pltpu.async_copy(ent_seg_hbm.at[idx_h], rows_h, sem)
    c2 = pltpu.async_copy(ent_seg_hbm.at[idx_t], rows_t, sem)
    c3 = pltpu.async_copy(rel_seg_hbm.at[idx_r], rows_r, sem)
    c1.wait()
    c2.wait()
    c3.wait()

    def body(i, carry):
        h_re = rows_h[i, pl.ds(0, L)]
        h_im = rows_h[i, pl.ds(L, L)]
        t_re = rows_t[i, pl.ds(0, L)]
        t_im = rows_t[i, pl.ds(L, L)]
        r_re = rows_r[i, pl.ds(0, L)]
        r_im = rows_r[i, pl.ds(L, L)]
        p = h_re * t_re + h_im * t_im
        q = h_re * t_im - h_im * t_re
        accbuf[pl.ds(i * ACC_STRIDE, L)] = r_re * p + r_im * q
        return carry

    lax.fori_loop(0, B_PER_W, body, 0)
    _transpose_reduce(accbuf, out_v, 0, lane, B_PER_W)
    pltpu.sync_copy(out_v, out_hbm.at[pl.ds(base, B_PER_W)])


@jax.jit
def _compl_ex(heads, rels, tails, entity_emb, rel_emb):
    mesh = plsc.VectorSubcoreMesh(
        core_axis_name="c", subcore_axis_name="s", num_cores=NC,
        num_subcores=NS)
    main = functools.partial(
        pl.kernel,
        out_type=jax.ShapeDtypeStruct((BATCH,), jnp.float32),
        mesh=mesh,
        compiler_params=pltpu.CompilerParams(needs_layout_passes=False),
        scratch_types=[
            [pltpu.VMEM((B_PER_W,), jnp.int32) for _ in range(3)],
            [[pltpu.VMEM((CHUNK, 128), jnp.float32) for _ in range(9)]
             for _ in range(2)],
            pltpu.VMEM((CHUNK * ACC_STRIDE,), jnp.float32),
            pltpu.VMEM((B_PER_W,), jnp.float32),
            [pltpu.SemaphoreType.DMA for _ in range(2)],
        ],
    )(_main_kernel)
    seg = functools.partial(
        pl.kernel,
        out_type=jax.ShapeDtypeStruct((BATCH,), jnp.float32),
        mesh=mesh,
        compiler_params=pltpu.CompilerParams(
            needs_layout_passes=False, use_tc_tiling_on_sc=False),
        scratch_types=[
            pltpu.VMEM((B_PER_W,), jnp.int32),
            pltpu.VMEM((B_PER_W,), jnp.int32),
            pltpu.VMEM((B_PER_W,), jnp.int32),
            pltpu.VMEM((B_PER_W, 2 * L), jnp.float32),
            pltpu.VMEM((B_PER_W, 2 * L), jnp.float32),
            pltpu.VMEM((B_PER_W, 2 * L), jnp.float32),
            pltpu.VMEM((B_PER_W * ACC_STRIDE,), jnp.float32),
            pltpu.VMEM((B_PER_W,), jnp.float32),
            pltpu.SemaphoreType.DMA,
        ],
    )(_seg_kernel)
    ent_seg = jnp.concatenate(
        [entity_emb[:, SEG:DIM], entity_emb[:, DIM + SEG:]], axis=1)
    rel_seg = jnp.concatenate(
        [rel_emb[:, SEG:DIM], rel_emb[:, DIM + SEG:]], axis=1)
    part1 = main(heads, rels, tails, entity_emb, rel_emb)
    part2 = seg(heads, rels, tails, ent_seg, rel_seg)
    return part1 + part2


def kernel(heads, rels, tails, entity_emb, rel_emb):
    return _compl_ex(
        heads.astype(jnp.int32),
        rels.astype(jnp.int32),
        tails.astype(jnp.int32),
        entity_emb.astype(jnp.float32),
        rel_emb.astype(jnp.float32),
    )


# speedup vs baseline: 1.0745x; 1.0062x over previous
import functools

import jax
import jax.numpy as jnp
from jax import lax
from jax.experimental import pallas as pl
from jax.experimental.pallas import tpu as pltpu
from jax.experimental.pallas import tpu_sc as plsc

NUM_ENTITIES = 100000
NUM_RELATIONS = 1000
DIM = 200
BATCH = 16384

NC, NS, L = 2, 16, 16
NW = NC * NS
B_PER_W = BATCH // NW
CHUNK = 32
NCHUNK = B_PER_W // CHUNK
ACC_STRIDE = L + 1
SEG = 184

_GROUPS = [
    (0, 0, 1, 72, False), (0, 16, 1, 88, False), (0, 32, 1, 104, False),
    (0, 40, 1, 112, True),
    (0, 56, 2, 0, False), (0, 72, 2, 16, False), (0, 88, 2, 32, False),
    (0, 104, 2, 48, False), (0, 112, 2, 56, True),
    (1, 0, 2, 72, False), (1, 16, 2, 88, False), (1, 32, 2, 104, False),
    (1, 40, 2, 112, True),
]


def _transpose_reduce(accbuf, out_v, out_base, lane, count):
    for k in range(0, count, L):
        base_idx = (lane + k) * ACC_STRIDE
        tot = jnp.zeros((L,), jnp.float32)
        for j in range(L):
            tot = tot + plsc.load_gather(accbuf, [base_idx + j])
        out_v[pl.ds(out_base + k, L)] = tot


def _main_kernel(heads_hbm, rels_hbm, tails_hbm, ent_hbm, rel_hbm,
                 out_hbm, idx_all, bufs_s, accbuf, out_v, sems):
    wid = lax.axis_index("s") * NC + lax.axis_index("c")
    base = wid * B_PER_W
    lane = lax.iota(jnp.int32, L)
    half_mask = lane >= L // 2
    srcs = (heads_hbm, tails_hbm, rels_hbm)
    tables = (ent_hbm, ent_hbm, rel_hbm)

    for t in range(3):
        pltpu.sync_copy(srcs[t].at[pl.ds(base, B_PER_W)], idx_all[t])

    def copies(g, slot):
        bufs = bufs_s[slot]
        return [
            pltpu.make_async_copy(
                tables[t].at[idx_all[t].at[pl.ds(g * CHUNK, CHUNK)],
                             pl.ds(128 * k, 128)],
                bufs[3 * t + k], sems[slot])
            for t in range(3) for k in range(3)
        ]

    def fire(g, slot):
        for c in copies(g, slot):
            c.start()

    def drain(g, slot):
        for c in copies(g, slot):
            c.wait()

    def score(g, slot):
        bufs = bufs_s[slot]

        def body(i, carry2):
            acc = jnp.zeros((L,), jnp.float32)
            for re_p, re_o, im_p, im_o, masked in _GROUPS:
                h_re = bufs[re_p][i, pl.ds(re_o, L)]
                h_im = bufs[im_p][i, pl.ds(im_o, L)]
                t_re = bufs[3 + re_p][i, pl.ds(re_o, L)]
                t_im = bufs[3 + im_p][i, pl.ds(im_o, L)]
                r_re = bufs[6 + re_p][i, pl.ds(re_o, L)]
                r_im = bufs[6 + im_p][i, pl.ds(im_o, L)]
                p = h_re * t_re + h_im * t_im
                q = h_re * t_im - h_im * t_re
                term = r_re * p + r_im * q
                if masked:
                    term = jnp.where(half_mask, term, 0.0)
                acc = acc + term
            accbuf[pl.ds(i * ACC_STRIDE, L)] = acc
            return carry2

        lax.fori_loop(0, CHUNK, body, 0)
        _transpose_reduce(accbuf, out_v, g * CHUNK, lane, CHUNK)

    fire(0, 0)

    def pair_body(g2, carry):
        g = 2 * g2
        drain(g, 0)
        fire(g + 1, 1)
        score(g, 0)
        drain(g + 1, 1)

        @pl.when(g + 2 < NCHUNK)
        def _():
            fire(g + 2, 0)

        score(g + 1, 1)
        return carry

    lax.fori_loop(0, NCHUNK // 2, pair_body, 0)
    pltpu.sync_copy(out_v, out_hbm.at[pl.ds(base, B_PER_W)])


def _seg_kernel(heads_hbm, rels_hbm, tails_hbm, ent_seg_hbm, rel_seg_hbm,
                out_hbm, idx_h, idx_t, idx_r, rows_h, rows_t, rows_r,
                accbuf, out_v, sem):
    wid = lax.axis_index("s") * NC + lax.axis_index("c")
    base = wid * B_PER_W
    lane = lax.iota(jnp.int32, L)

    pltpu.sync_copy(heads_hbm.at[pl.ds(base, B_PER_W)], idx_h)
    pltpu.sync_copy(tails_hbm.at[pl.ds(base, B_PER_W)], idx_t)
    pltpu.sync_copy(rels_hbm.at[pl.ds(base, B_PER_W)], idx_r)
    c1 = pltpu.async_copy(ent_seg_hbm.at[idx_h], rows_h, sem)
    c2 = pltpu.async_copy(ent_seg_hbm.at[idx_t], rows_t, sem)
    c3 = pltpu.async_copy(rel_seg_hbm.at[idx_r], rows_r, sem)
    c1.wait()
    c2.wait()
    c3.wait()

    def body(i, carry):
        h_re = rows_h[i, pl.ds(0, L)]
        h_im = rows_h[i, pl.ds(L, L)]
        t_re = rows_t[i, pl.ds(0, L)]
        t_im = rows_t[i, pl.ds(L, L)]
        r_re = rows_r[i, pl.ds(0, L)]
        r_im = rows_r[i, pl.ds(L, L)]
        p = h_re * t_re + h_im * t_im
        q = h_re * t_im - h_im * t_re
        accbuf[pl.ds(i * ACC_STRIDE, L)] = r_re * p + r_im * q
        return carry

    lax.fori_loop(0, B_PER_W, body, 0)
    _transpose_reduce(accbuf, out_v, 0, lane, B_PER_W)
    pltpu.sync_copy(out_v, out_hbm.at[pl.ds(base, B_PER_W)])


@jax.jit
def _compl_ex(heads, rels, tails, entity_emb, rel_emb):
    mesh = plsc.VectorSubcoreMesh(
        core_axis_name="c", subcore_axis_name="s", num_cores=NC,
        num_subcores=NS)
    main = functools.partial(
        pl.kernel,
        out_type=jax.ShapeDtypeStruct((BATCH,), jnp.float32),
        mesh=mesh,
        compiler_params=pltpu.CompilerParams(needs_layout_passes=False),
        scratch_types=[
            [pltpu.VMEM((B_PER_W,), jnp.int32) for _ in range(3)],
            [[pltpu.VMEM((CHUNK, 128), jnp.float32) for _ in range(9)]
             for _ in range(2)],
            pltpu.VMEM((CHUNK * ACC_STRIDE,), jnp.float32),
            pltpu.VMEM((B_PER_W,), jnp.float32),
            [pltpu.SemaphoreType.DMA for _ in range(2)],
        ],
    )(_main_kernel)
    seg = functools.partial(
        pl.kernel,
        out_type=jax.ShapeDtypeStruct((BATCH,), jnp.float32),
        mesh=mesh,
        compiler_params=pltpu.CompilerParams(
            needs_layout_passes=False, use_tc_tiling_on_sc=False),
        scratch_types=[
            pltpu.VMEM((B_PER_W,), jnp.int32),
            pltpu.VMEM((B_PER_W,), jnp.int32),
            pltpu.VMEM((B_PER_W,), jnp.int32),
            pltpu.VMEM((B_PER_W, 2 * L), jnp.float32),
            pltpu.VMEM((B_PER_W, 2 * L), jnp.float32),
            pltpu.VMEM((B_PER_W, 2 * L), jnp.float32),
            pltpu.VMEM((B_PER_W * ACC_STRIDE,), jnp.float32),
            pltpu.VMEM((B_PER_W,), jnp.float32),
            pltpu.SemaphoreType.DMA,
        ],
    )(_seg_kernel)
    ent_seg = jnp.concatenate(
        [entity_emb[:, SEG:DIM], entity_emb[:, DIM + SEG:]], axis=1)
    rel_seg = jnp.concatenate(
        [rel_emb[:, SEG:DIM], rel_emb[:, DIM + SEG:]], axis=1)
    part2 = seg(heads, rels, tails, ent_seg, rel_seg)
    part1 = main(heads, rels, tails, entity_emb, rel_emb)
    return part1 + part2


def kernel(heads, rels, tails, entity_emb, rel_emb):
    return _compl_ex(
        heads.astype(jnp.int32),
        rels.astype(jnp.int32),
        tails.astype(jnp.int32),
        entity_emb.astype(jnp.float32),
        rel_emb.astype(jnp.float32),
    )
